# Initial kernel scaffold; baseline (speedup 1.0000x reference)
import jax
import jax.numpy as jnp
from jax.experimental import pallas as pl


def kernel(x, edge_index, W_proj, b_proj, basis1, coeff1, bias1, basis2, coeff2, bias2, prompt_w):
    raise NotImplementedError("write your pallas kernel here")


# speedup vs baseline: 3.0503x; 3.0503x over previous
import jax
import jax.numpy as jnp
from jax import lax
from jax.experimental import pallas as pl
from jax.experimental.pallas import tpu as pltpu
from jax.experimental.pallas import tpu_sc as plsc

N = 10000
D = 128
E = 320000
R = 4
NC = 2
NS = 16
CH = 128
NW = 40
SLAB = 632
NPAD = NS * SLAB

PER_TILE_REL = E // R // NS
PER_TILE_ALL = E // (NC * NS)

def _mesh():
  return plsc.VectorSubcoreMesh(
      core_axis_name="c", subcore_axis_name="s", num_cores=NC, num_subcores=NS)


def _pipe(h_hbm, src_v, dst_v, nch, gbuf0, gbuf1, acc, gsem, ssem):
  def one_iter(j, gcur, gnext, prev_pred, next_pred):
    pltpu.make_async_copy(h_hbm.at[src_v.at[j]], gcur, gsem).wait()

    def wait_prev():
      pltpu.make_async_copy(gnext, acc.at[dst_v.at[j - 1]], ssem).wait()
    if prev_pred is True:
      wait_prev()
    elif prev_pred is not False:
      pl.when(prev_pred)(wait_prev)

    pltpu.make_async_copy(gcur, acc.at[dst_v.at[j]], ssem).start(add=True)

    def issue_next():
      pltpu.make_async_copy(h_hbm.at[src_v.at[j + 1]], gnext, gsem).start()
    if next_pred is True:
      issue_next()
    elif next_pred is not False:
      pl.when(next_pred)(issue_next)

  pltpu.make_async_copy(h_hbm.at[src_v.at[0]], gbuf0, gsem).start()

  npair = nch // 2

  def pair(p, carry):
    j0 = p * 2
    one_iter(j0, gbuf0, gbuf1, prev_pred=(p >= 1), next_pred=True)
    one_iter(j0 + 1, gbuf1, gbuf0, prev_pred=True, next_pred=(p < npair - 1))
    return carry

  lax.fori_loop(0, npair, pair, 0)

  last = nch - 1
  lastbuf = gbuf0 if last % 2 == 0 else gbuf1
  pltpu.make_async_copy(lastbuf, acc.at[dst_v.at[last]], ssem).wait()


def _sc_seg_kernel(n_rows, windows_per_phase, n_phases):
  n_out = R if n_phases == 2 else NC
  out_type = jax.ShapeDtypeStruct((n_out, NPAD, D), jnp.float32)
  scratch = [
      pltpu.VMEM((NW, CH), jnp.int32),
      pltpu.VMEM((NW, CH), jnp.int32),
      pltpu.VMEM((CH, D), jnp.float32),
      pltpu.VMEM((CH, D), jnp.float32),
      pltpu.VMEM_SHARED((NPAD, D), jnp.float32),
      pltpu.SemaphoreType.DMA,
      pltpu.SemaphoreType.DMA,
  ]

  def body(src_hbm, dst_hbm, h_hbm, zeros_hbm, s_out,
           src_v, dst_v, gbuf0, gbuf1, acc, gsem, ssem):
    c = lax.axis_index("c")
    s = lax.axis_index("s")
    slab = pl.ds(s * SLAB, SLAB)
    for ph in range(n_phases):
      out_idx = c * n_phases + ph
      pltpu.sync_copy(zeros_hbm, acc.at[slab])
      plsc.subcore_barrier()
      for w in range(windows_per_phase):
        win = pl.ds(w * NW, NW)
        pltpu.sync_copy(src_hbm.at[out_idx, s, win], src_v)
        pltpu.sync_copy(dst_hbm.at[out_idx, s, win], dst_v)
        _pipe(h_hbm, src_v, dst_v, NW, gbuf0, gbuf1, acc, gsem, ssem)
      plsc.subcore_barrier()
      pltpu.sync_copy(acc.at[slab], s_out.at[out_idx, slab])
      plsc.subcore_barrier()

  return pl.kernel(body, out_type, mesh=_mesh(), scratch_types=scratch)


def _sc_deg_kernel():
  out_type = jax.ShapeDtypeStruct((R, NPAD, D), jnp.float32)
  scratch = [
      pltpu.VMEM((NW, CH), jnp.int32),
      pltpu.VMEM((CH, D), jnp.float32),
      pltpu.VMEM_SHARED((NPAD, D), jnp.float32),
      pltpu.SemaphoreType.DMA,
  ]

  def body(dst_hbm, zeros_hbm, ones_hbm, deg_out,
           dst_v, ones_v, dacc, dsem):
    c = lax.axis_index("c")
    s = lax.axis_index("s")
    slab = pl.ds(s * SLAB, SLAB)
    pltpu.sync_copy(ones_hbm, ones_v)
    for rr in range(R // NC):
      rel = c * (R // NC) + rr
      pltpu.sync_copy(zeros_hbm, dacc.at[slab])
      plsc.subcore_barrier()
      pltpu.sync_copy(dst_hbm.at[rel, s], dst_v)

      def step(j, carry):
        def wait_prev():
          pltpu.make_async_copy(ones_v, dacc.at[dst_v.at[j - 1]], dsem).wait()
        pl.when(j >= 1)(wait_prev)
        pltpu.make_async_copy(ones_v, dacc.at[dst_v.at[j]], dsem).start(
            add=True)
        return carry

      lax.fori_loop(0, NW, step, 0)
      pltpu.make_async_copy(ones_v, dacc.at[dst_v.at[NW - 1]], dsem).wait()
      plsc.subcore_barrier()
      pltpu.sync_copy(dacc.at[slab], deg_out.at[rel, slab])
      plsc.subcore_barrier()

  return pl.kernel(body, out_type, mesh=_mesh(), scratch_types=scratch)


def _proj_body(x_ref, w_ref, b_ref, o_ref):
  o_ref[...] = (jnp.dot(x_ref[...], w_ref[...],
                        preferred_element_type=jnp.float32) + b_ref[...])


def _proj(x, W, b2d):
  br = 1000
  return pl.pallas_call(
      _proj_body,
      grid=(N // br,),
      in_specs=[
          pl.BlockSpec((br, D), lambda i: (i, 0)),
          pl.BlockSpec((D, D), lambda i: (0, 0)),
          pl.BlockSpec((1, D), lambda i: (0, 0)),
      ],
      out_specs=pl.BlockSpec((br, D), lambda i: (i, 0)),
      out_shape=jax.ShapeDtypeStruct((N, D), jnp.float32),
  )(x, W, b2d)


def _make_combine_body(br, prompt):
  def body(coeff_ref, s_ref, deg_ref, basis_ref, bias_ref, pw_ref, o_ref):
    t0 = jnp.zeros((br, D), jnp.float32)
    t1 = jnp.zeros((br, D), jnp.float32)
    for r in range(R):
      dinv = 1.0 / jnp.maximum(deg_ref[r, :, 0:1], 1.0)
      u = s_ref[r] * dinv
      t0 = t0 + coeff_ref[r, 0] * u
      t1 = t1 + coeff_ref[r, 1] * u
    h = (jnp.dot(t0, basis_ref[0], preferred_element_type=jnp.float32)
         + jnp.dot(t1, basis_ref[1], preferred_element_type=jnp.float32)
         + bias_ref[...])
    h = jnp.maximum(h, 0.0)
    if prompt:
      z = h * pw_ref[...]
      h = jnp.where(z > 0.0, z, jnp.exp(z) - 1.0)
    o_ref[...] = h
  return body


def _combine(s, deg, basis, coeff, bias2d, prompt_w):
  br = 632
  prompt = prompt_w is not None
  if prompt_w is None:
    prompt_w = bias2d
  return pl.pallas_call(
      _make_combine_body(br, prompt),
      grid=(NPAD // br,),
      in_specs=[
          pl.BlockSpec(memory_space=pltpu.SMEM),
          pl.BlockSpec((R, br, D), lambda i: (0, i, 0)),
          pl.BlockSpec((R, br, D), lambda i: (0, i, 0)),
          pl.BlockSpec((2, D, D), lambda i: (0, 0, 0)),
          pl.BlockSpec((1, D), lambda i: (0, 0)),
          pl.BlockSpec((1, D), lambda i: (0, 0)),
      ],
      out_specs=pl.BlockSpec((br, D), lambda i: (i, 0)),
      out_shape=jax.ShapeDtypeStruct((NPAD, D), jnp.float32),
  )(coeff, s, deg, basis, bias2d, prompt_w)


def _sum_partials_body(p_ref, o_ref):
  o_ref[...] = p_ref[0] + p_ref[1]


def _sum_partials(p):
  br = 632
  return pl.pallas_call(
      _sum_partials_body,
      grid=(NPAD // br,),
      in_specs=[pl.BlockSpec((NC, br, D), lambda i: (0, i, 0))],
      out_specs=pl.BlockSpec((br, D), lambda i: (i, 0)),
      out_shape=jax.ShapeDtypeStruct((NPAD, D), jnp.float32),
  )(p)


def kernel(x, edge_index, W_proj, b_proj, basis1, coeff1, bias1,
           basis2, coeff2, bias2, prompt_w):
  ei = edge_index.astype(jnp.int32)

  pad_rel = NW * CH - PER_TILE_REL
  src_r = ei[0].reshape(R, NS, PER_TILE_REL)
  dst_r = ei[1].reshape(R, NS, PER_TILE_REL)
  src_rp = jnp.pad(src_r, ((0, 0), (0, 0), (0, pad_rel))
                   ).reshape(R, NS, NW, CH)
  dst_rp = jnp.pad(dst_r, ((0, 0), (0, 0), (0, pad_rel)),
                   constant_values=NPAD - 1).reshape(R, NS, NW, CH)

  pad_all = 2 * NW * CH - PER_TILE_ALL
  src_a = ei[0].reshape(NC, NS, PER_TILE_ALL)
  dst_a = ei[1].reshape(NC, NS, PER_TILE_ALL)
  src_ap = jnp.pad(src_a, ((0, 0), (0, 0), (0, pad_all))
                   ).reshape(NC, NS, 2 * NW, CH)
  dst_ap = jnp.pad(dst_a, ((0, 0), (0, 0), (0, pad_all)),
                   constant_values=NPAD - 1).reshape(NC, NS, 2 * NW, CH)

  zeros = jnp.zeros((SLAB, D), jnp.float32)
  ones = jnp.ones((CH, D), jnp.float32)

  h0 = _proj(x, W_proj, b_proj.reshape(1, D))

  deg = _sc_deg_kernel()(dst_rp, zeros, ones)
  s1 = _sc_seg_kernel(N, 1, 2)(src_rp, dst_rp, h0, zeros)
  h1 = _combine(s1, deg, basis1, coeff1, bias1.reshape(1, D), None)

  s2 = _sc_seg_kernel(NPAD, 1, 2)(src_rp, dst_rp, h1, zeros)
  hp = _combine(s2, deg, basis2, coeff2, bias2.reshape(1, D),
                prompt_w.reshape(1, D))

  part = _sc_seg_kernel(NPAD, 2, 1)(src_ap, dst_ap, hp, zeros)
  out = _sum_partials(part)
  return out[:N]
